# baseline (speedup 1.0000x reference)
import functools

import jax
import jax.numpy as jnp
from jax import lax
from jax.experimental import pallas as pl
from jax.experimental.pallas import tpu as pltpu
from jax.experimental.pallas import tpu_sc as plsc

B = 4096
F = 26
D = 32
NC = 2
NS = 16
NW = NC * NS
BPW = B // NW
L = 16
RPL = 128 // D


def _sc_body(emb_hbm, lin_hbm, idx_hbm, s_out, q_out, l_out,
             idx_v, sup_v, ebuf, lbuf, acc_s, acc_q, acc_l,
             esem0, esem1, lsem):
    c = lax.axis_index("c")
    s = lax.axis_index("s")
    wid = s * NC + c

    pltpu.sync_copy(idx_hbm.at[wid], idx_v)
    for f in range(F):
        for k in range(BPW // L):
            sl = pl.ds(k * L, L)
            sup_v[f, sl] = lax.shift_right_logical(idx_v[f, sl], 2)

    ldescs = [pltpu.async_copy(lin_hbm.at[idx_v.at[f]], lbuf.at[f], lsem)
              for f in range(F)]

    zvec = jnp.zeros((L,), jnp.float32)

    def zero_body(r, carry):
        for half in range(D // L):
            asl = pl.ds(half * L, L)
            acc_s[r, asl] = zvec
            acc_q[r, asl] = zvec
        return carry

    lax.fori_loop(0, BPW, zero_body, 0)

    def field_acc(f, jb):
        def grp_body(g, carry):
            r0 = g * L
            offv = (idx_v[f, pl.ds(r0, L)] & (RPL - 1)) * D
            for i in range(L):
                off = offv[i]
                r = r0 + i
                for half in range(D // L):
                    sl = pl.ds(off + half * L, L)
                    asl = pl.ds(half * L, L)
                    v = ebuf[jb, r, sl]
                    acc_s[r, asl] += v
                    acc_q[r, asl] += v * v
            return carry
        lax.fori_loop(0, BPW // L, grp_body, 0)

    pltpu.async_copy(emb_hbm.at[sup_v.at[0]], ebuf.at[0], esem0)
    pltpu.async_copy(emb_hbm.at[sup_v.at[1]], ebuf.at[1], esem1)

    def pair_body(t, carry):
        f0 = 2 * t
        pltpu.make_async_copy(emb_hbm.at[sup_v.at[f0]], ebuf.at[0],
                              esem0).wait()
        field_acc(f0, 0)

        @pl.when(f0 + 2 < F)
        def _():
            pltpu.async_copy(emb_hbm.at[sup_v.at[f0 + 2]], ebuf.at[0], esem0)

        pltpu.make_async_copy(emb_hbm.at[sup_v.at[f0 + 1]], ebuf.at[1],
                              esem1).wait()
        field_acc(f0 + 1, 1)

        @pl.when(f0 + 3 < F)
        def _():
            pltpu.async_copy(emb_hbm.at[sup_v.at[f0 + 3]], ebuf.at[1], esem1)

        return carry

    lax.fori_loop(0, F // 2, pair_body, 0)

    for d_ in ldescs:
        d_.wait()
    for k in range(BPW // L):
        sl = pl.ds(k * L, L)
        vs = [lbuf[f, sl] for f in range(F)]
        while len(vs) > 1:
            vs = [vs[i] + vs[i + 1] for i in range(0, len(vs) - 1, 2)] + (
                [vs[-1]] if len(vs) % 2 else [])
        acc_l[sl] = vs[0]

    base = wid * BPW
    pltpu.sync_copy(acc_s, s_out.at[pl.ds(base, BPW)])
    pltpu.sync_copy(acc_q, q_out.at[pl.ds(base, BPW)])
    pltpu.sync_copy(acc_l, l_out.at[pl.ds(base, BPW)])


def _sc_reduce(emb_lines, lin_flat, idx_arr):
    mesh = plsc.VectorSubcoreMesh(core_axis_name="c", subcore_axis_name="s")
    fn = functools.partial(
        pl.kernel,
        mesh=mesh,
        out_type=[
            jax.ShapeDtypeStruct((B, D), jnp.float32),
            jax.ShapeDtypeStruct((B, D), jnp.float32),
            jax.ShapeDtypeStruct((B,), jnp.float32),
        ],
        scratch_types=[
            pltpu.VMEM((F, BPW), jnp.int32),
            pltpu.VMEM((F, BPW), jnp.int32),
            pltpu.VMEM((2, BPW, 128), jnp.float32),
            pltpu.VMEM((F, BPW), jnp.float32),
            pltpu.VMEM((BPW, D), jnp.float32),
            pltpu.VMEM((BPW, D), jnp.float32),
            pltpu.VMEM((BPW,), jnp.float32),
            pltpu.SemaphoreType.DMA,
            pltpu.SemaphoreType.DMA,
            pltpu.SemaphoreType.DMA,
        ],
    )(_sc_body)
    return fn(emb_lines, lin_flat, idx_arr)


def _tc_body(s_ref, q_ref, l_ref, w1, b1, w2, b2, w3, b3, o_ref):
    sv = s_ref[...]
    qv = q_ref[...]
    bi = 0.5 * (sv * sv - qv)
    h = jnp.maximum(jnp.dot(bi, w1[...], preferred_element_type=jnp.float32)
                    + b1[...], 0.0)
    h = jnp.maximum(jnp.dot(h, w2[...], preferred_element_type=jnp.float32)
                    + b2[...], 0.0)
    deep = jnp.dot(h, w3[...], preferred_element_type=jnp.float32)
    o_ref[...] = l_ref[...] + deep + b3[...]


def _tc_mlp(S, Q, Lsum, W1, b1, W2, b2, W3, b3):
    out = pl.pallas_call(
        _tc_body,
        out_shape=jax.ShapeDtypeStruct((B, 1), jnp.float32),
    )(S, Q, Lsum.reshape(B, 1), W1, b1[None], W2, b2[None], W3, b3[None])
    return out.reshape(B)


def kernel(features, emb_table, lin_table, W1, b1, W2, b2, W3, b3):
    idx_arr = (features.astype(jnp.int32).T
               .reshape(F, NW, BPW).transpose(1, 0, 2))
    emb_lines = emb_table.reshape(emb_table.shape[0] // RPL, 128)
    lin_flat = lin_table.reshape(-1)
    S, Q, Lsum = _sc_reduce(emb_lines, lin_flat, idx_arr)
    return _tc_mlp(S, Q, Lsum, W1, b1, W2, b2, W3, b3)


# speedup vs baseline: 1.3489x; 1.3489x over previous
import functools

import jax
import jax.numpy as jnp
from jax import lax
from jax.experimental import pallas as pl
from jax.experimental.pallas import tpu as pltpu
from jax.experimental.pallas import tpu_sc as plsc

B = 4096
F = 26
D = 32
NC = 2
NS = 16
NW = NC * NS
BPW = B // NW
L = 16


def _tree_sum(vs):
    while len(vs) > 1:
        vs = [vs[i] + vs[i + 1] for i in range(0, len(vs) - 1, 2)] + (
            [vs[-1]] if len(vs) % 2 else [])
    return vs[0]


def _sc_body(emb_hbm, lin_hbm, idx_hbm, lidx_hbm, s_out, q_out, l_out,
             idx_v, lidx_v, rbuf, lbuf, acc_s, acc_q, acc_l,
             esem0, esem1, lsem):
    c = lax.axis_index("c")
    s = lax.axis_index("s")
    wid = s * NC + c

    pltpu.sync_copy(idx_hbm.at[wid], idx_v)
    pltpu.sync_copy(lidx_hbm.at[wid], lidx_v)

    ldescs = [pltpu.async_copy(lin_hbm.at[lidx_v.at[f]], lbuf.at[f], lsem)
              for f in range(F)]

    def fire_row(r, par, sem):
        va = idx_v[r, pl.ds(0, L)]
        vb = idx_v[r, pl.ds(F - L, L)]
        for f in range(F):
            ridx = va[f] if f < L else vb[f - (F - L)]
            pltpu.async_copy(emb_hbm.at[pl.ds(ridx, 1)],
                             rbuf.at[par, pl.ds(f, 1)], sem)

    def acc_row(r, par, sem):
        for _ in range(F):
            pltpu.make_async_copy(emb_hbm.at[pl.ds(0, 1)],
                                  rbuf.at[par, pl.ds(0, 1)], sem).wait()
        for half in range(D // L):
            sl = pl.ds(half * L, L)
            vs = [rbuf[par, f, sl] for f in range(F)]
            acc_s[r, sl] = _tree_sum(vs)
            acc_q[r, sl] = _tree_sum([v * v for v in vs])

    fire_row(0, 0, esem0)

    def pair_body(t, carry):
        r0 = 2 * t
        fire_row(r0 + 1, 1, esem1)
        acc_row(r0, 0, esem0)

        @pl.when(r0 + 2 < BPW)
        def _():
            fire_row(r0 + 2, 0, esem0)

        acc_row(r0 + 1, 1, esem1)
        return carry

    lax.fori_loop(0, BPW // 2, pair_body, 0)

    for d_ in ldescs:
        d_.wait()
    for k in range(BPW // L):
        sl = pl.ds(k * L, L)
        acc_l[sl] = _tree_sum([lbuf[f, sl] for f in range(F)])

    base = wid * BPW
    pltpu.sync_copy(acc_s, s_out.at[pl.ds(base, BPW)])
    pltpu.sync_copy(acc_q, q_out.at[pl.ds(base, BPW)])
    pltpu.sync_copy(acc_l, l_out.at[pl.ds(base, BPW)])


def _sc_reduce(emb_table, lin_flat, idx_arr, lidx_arr):
    mesh = plsc.VectorSubcoreMesh(core_axis_name="c", subcore_axis_name="s")
    fn = functools.partial(
        pl.kernel,
        mesh=mesh,
        out_type=[
            jax.ShapeDtypeStruct((B, D), jnp.float32),
            jax.ShapeDtypeStruct((B, D), jnp.float32),
            jax.ShapeDtypeStruct((B,), jnp.float32),
        ],
        scratch_types=[
            pltpu.VMEM((BPW, F), jnp.int32),
            pltpu.VMEM((F, BPW), jnp.int32),
            pltpu.VMEM((2, F, D), jnp.float32),
            pltpu.VMEM((F, BPW), jnp.float32),
            pltpu.VMEM((BPW, D), jnp.float32),
            pltpu.VMEM((BPW, D), jnp.float32),
            pltpu.VMEM((BPW,), jnp.float32),
            pltpu.SemaphoreType.DMA,
            pltpu.SemaphoreType.DMA,
            pltpu.SemaphoreType.DMA,
        ],
    )(_sc_body)
    return fn(emb_table, lin_flat, idx_arr, lidx_arr)


def _tc_body(s_ref, q_ref, l_ref, w1, b1, w2, b2, w3, b3, o_ref):
    sv = s_ref[...]
    qv = q_ref[...]
    bi = 0.5 * (sv * sv - qv)
    h = jnp.maximum(jnp.dot(bi, w1[...], preferred_element_type=jnp.float32)
                    + b1[...], 0.0)
    h = jnp.maximum(jnp.dot(h, w2[...], preferred_element_type=jnp.float32)
                    + b2[...], 0.0)
    deep = jnp.dot(h, w3[...], preferred_element_type=jnp.float32)
    o_ref[...] = l_ref[...] + deep + b3[...]


def _tc_mlp(S, Q, Lsum, W1, b1, W2, b2, W3, b3):
    out = pl.pallas_call(
        _tc_body,
        out_shape=jax.ShapeDtypeStruct((B, 1), jnp.float32),
    )(S, Q, Lsum.reshape(B, 1), W1, b1[None], W2, b2[None], W3, b3[None])
    return out.reshape(B)


def kernel(features, emb_table, lin_table, W1, b1, W2, b2, W3, b3):
    feats = features.astype(jnp.int32)
    idx_arr = feats.reshape(NW, BPW, F)
    lidx_arr = feats.T.reshape(F, NW, BPW).transpose(1, 0, 2)
    lin_flat = lin_table.reshape(-1)
    S, Q, Lsum = _sc_reduce(emb_table, lin_flat, idx_arr, lidx_arr)
    return _tc_mlp(S, Q, Lsum, W1, b1, W2, b2, W3, b3)


# speedup vs baseline: 1.4275x; 1.0583x over previous
import functools

import jax
import jax.numpy as jnp
from jax import lax
from jax.experimental import pallas as pl
from jax.experimental.pallas import tpu as pltpu
from jax.experimental.pallas import tpu_sc as plsc

B = 4096
F = 26
D = 32
NC = 2
NS = 16
NW = NC * NS
BPW = B // NW
L = 16


def _tree_sum(vs):
    while len(vs) > 1:
        vs = [vs[i] + vs[i + 1] for i in range(0, len(vs) - 1, 2)] + (
            [vs[-1]] if len(vs) % 2 else [])
    return vs[0]


def _sc_body(emb_hbm, lin_hbm, idx_hbm, lidx_hbm, s_out, q_out, l_out,
             idx_v, lidx_v, rbuf, lbuf, acc_s, acc_q, acc_l,
             esem0, esem1, esem2, esem3, lsem):
    c = lax.axis_index("c")
    s = lax.axis_index("s")
    wid = s * NC + c

    pltpu.sync_copy(idx_hbm.at[wid], idx_v)
    pltpu.sync_copy(lidx_hbm.at[wid], lidx_v)

    ldescs = [pltpu.async_copy(lin_hbm.at[lidx_v.at[f]], lbuf.at[f], lsem)
              for f in range(F)]

    def fire_row(r, par, sem):
        va = idx_v[r, pl.ds(0, L)]
        vb = idx_v[r, pl.ds(F - L, L)]
        for f in range(F):
            ridx = va[f] if f < L else vb[f - (F - L)]
            pltpu.async_copy(emb_hbm.at[pl.ds(ridx, 1)],
                             rbuf.at[par, pl.ds(f, 1)], sem)

    def acc_row(r, par, sem):
        for _ in range(F):
            pltpu.make_async_copy(emb_hbm.at[pl.ds(0, 1)],
                                  rbuf.at[par, pl.ds(0, 1)], sem).wait()
        for half in range(D // L):
            sl = pl.ds(half * L, L)
            ps = [None] * 4
            pq = [None] * 4
            for f in range(F):
                v = rbuf[par, f, sl]
                k = f % 4
                ps[k] = v if ps[k] is None else ps[k] + v
                q = v * v
                pq[k] = q if pq[k] is None else pq[k] + q
            acc_s[r, sl] = _tree_sum(ps)
            acc_q[r, sl] = _tree_sum(pq)

    sems = (esem0, esem1, esem2, esem3)
    for r in range(3):
        fire_row(r, r, sems[r])

    def quad_body(t, carry):
        r0 = 4 * t
        for j in range(4):
            r = r0 + j

            @pl.when(r + 3 < BPW)
            def _():
                fire_row(r + 3, (j + 3) % 4, sems[(j + 3) % 4])

            acc_row(r, j, sems[j])
        return carry

    lax.fori_loop(0, BPW // 4, quad_body, 0)

    for d_ in ldescs:
        d_.wait()
    for k in range(BPW // L):
        sl = pl.ds(k * L, L)
        acc_l[sl] = _tree_sum([lbuf[f, sl] for f in range(F)])

    base = wid * BPW
    pltpu.sync_copy(acc_s, s_out.at[pl.ds(base, BPW)])
    pltpu.sync_copy(acc_q, q_out.at[pl.ds(base, BPW)])
    pltpu.sync_copy(acc_l, l_out.at[pl.ds(base, BPW)])


def _sc_reduce(emb_table, lin_flat, idx_arr, lidx_arr):
    mesh = plsc.VectorSubcoreMesh(core_axis_name="c", subcore_axis_name="s")
    fn = functools.partial(
        pl.kernel,
        mesh=mesh,
        out_type=[
            jax.ShapeDtypeStruct((B, D), jnp.float32),
            jax.ShapeDtypeStruct((B, D), jnp.float32),
            jax.ShapeDtypeStruct((B,), jnp.float32),
        ],
        scratch_types=[
            pltpu.VMEM((BPW, F), jnp.int32),
            pltpu.VMEM((F, BPW), jnp.int32),
            pltpu.VMEM((4, F, D), jnp.float32),
            pltpu.VMEM((F, BPW), jnp.float32),
            pltpu.VMEM((BPW, D), jnp.float32),
            pltpu.VMEM((BPW, D), jnp.float32),
            pltpu.VMEM((BPW,), jnp.float32),
            pltpu.SemaphoreType.DMA,
            pltpu.SemaphoreType.DMA,
            pltpu.SemaphoreType.DMA,
            pltpu.SemaphoreType.DMA,
            pltpu.SemaphoreType.DMA,
        ],
    )(_sc_body)
    return fn(emb_table, lin_flat, idx_arr, lidx_arr)


def _tc_body(s_ref, q_ref, l_ref, w1, b1, w2, b2, w3, b3, o_ref):
    sv = s_ref[...]
    qv = q_ref[...]
    bi = 0.5 * (sv * sv - qv)
    h = jnp.maximum(jnp.dot(bi, w1[...], preferred_element_type=jnp.float32)
                    + b1[...], 0.0)
    h = jnp.maximum(jnp.dot(h, w2[...], preferred_element_type=jnp.float32)
                    + b2[...], 0.0)
    deep = jnp.dot(h, w3[...], preferred_element_type=jnp.float32)
    o_ref[...] = l_ref[...] + deep + b3[...]


def _tc_mlp(S, Q, Lsum, W1, b1, W2, b2, W3, b3):
    out = pl.pallas_call(
        _tc_body,
        out_shape=jax.ShapeDtypeStruct((B, 1), jnp.float32),
    )(S, Q, Lsum.reshape(B, 1), W1, b1[None], W2, b2[None], W3, b3[None])
    return out.reshape(B)


def kernel(features, emb_table, lin_table, W1, b1, W2, b2, W3, b3):
    feats = features.astype(jnp.int32)
    idx_arr = feats.reshape(NW, BPW, F)
    lidx_arr = feats.T.reshape(F, NW, BPW).transpose(1, 0, 2)
    lin_flat = lin_table.reshape(-1)
    S, Q, Lsum = _sc_reduce(emb_table, lin_flat, idx_arr, lidx_arr)
    return _tc_mlp(S, Q, Lsum, W1, b1, W2, b2, W3, b3)
